# baseline (speedup 1.0000x reference)
import functools

import jax
import jax.numpy as jnp
from jax import lax
from jax.experimental import pallas as pl
from jax.experimental.pallas import tpu as pltpu
from jax.experimental.pallas import tpu_sc as plsc

NC, NS, L = 2, 16, 16
NW = NC * NS


@functools.partial(jax.jit, static_argnums=(2, 3))
def _sc_gather(table_t, idx, heads, stride):
    win2, win2b = idx.shape
    n = win2 * win2b
    per_tile_rows = win2 // NW
    b = win2b
    chunks = per_tile_rows
    groups = b // L
    hb = heads * b

    mesh = plsc.VectorSubcoreMesh(core_axis_name="c", subcore_axis_name="s")

    @functools.partial(
        pl.kernel,
        out_type=jax.ShapeDtypeStruct((heads, win2, win2b), jnp.float32),
        mesh=mesh,
        compiler_params=pltpu.CompilerParams(needs_layout_passes=False),
        scratch_types=[
            pltpu.VMEM((heads * stride,), jnp.float32),
            pltpu.VMEM((2 * b,), jnp.int32),
            pltpu.VMEM((2 * hb,), jnp.float32),
            pltpu.SemaphoreType.DMA,
            pltpu.SemaphoreType.DMA,
            pltpu.SemaphoreType.DMA,
            pltpu.SemaphoreType.DMA,
            pltpu.SemaphoreType.DMA,
        ],
    )
    def k(tab_hbm, idx_hbm, out_hbm, tab_v, idx_v, out_v,
          sem_i0, sem_i1, sem_o0, sem_o1, sem_t):
        wid = lax.axis_index("s") * NC + lax.axis_index("c")
        row0 = wid * per_tile_rows
        for h in range(heads):
            pltpu.async_copy(tab_hbm.at[h],
                             tab_v.at[pl.ds(h * stride, stride)], sem_t)

        def idx_fetch(c, p, sem):
            pltpu.async_copy(idx_hbm.at[row0 + c],
                             idx_v.at[pl.ds(p * b, b)], sem)

        def drain_out(p, sem):
            for h in range(heads):
                pltpu.make_async_copy(out_hbm.at[0, 0],
                                      out_v.at[pl.ds(p * hb + h * b, b)],
                                      sem).wait()

        def do_chunk(c, p, sem_i, sem_o, first, last):
            pltpu.make_async_copy(idx_hbm.at[0],
                                  idx_v.at[pl.ds(p * b, b)], sem_i).wait()
            @pl.when(jnp.logical_not(first))
            def _():
                drain_out(p, sem_o)

            obase = p * hb
            ibase = p * b

            @plsc.parallel_loop(0, groups, unroll=4)
            def _(i):
                iv = idx_v[pl.ds(ibase + i * L, L)]
                for h in range(heads):
                    out_v[pl.ds(obase + h * b + i * L, L)] = (
                        plsc.load_gather(
                            tab_v.at[pl.ds(h * stride, stride)], [iv]))

            for h in range(heads):
                pltpu.async_copy(out_v.at[pl.ds(obase + h * b, b)],
                                 out_hbm.at[h, row0 + c], sem_o)

            @pl.when(jnp.logical_not(last))
            def _():
                idx_fetch(c + 2, p, sem_i)

        idx_fetch(0, 0, sem_i0)
        idx_fetch(1, 1, sem_i1)
        for h in range(heads):
            pltpu.make_async_copy(tab_hbm.at[h],
                                  tab_v.at[pl.ds(h * stride, stride)],
                                  sem_t).wait()

        n2 = chunks // 2

        @pl.loop(0, n2)
        def _(c2):
            c = 2 * c2
            do_chunk(c, 0, sem_i0, sem_o0, c2 == 0, c2 == n2 - 1)
            do_chunk(c + 1, 1, sem_i1, sem_o1, c2 == 0, c2 == n2 - 1)

        drain_out(0, sem_o0)
        drain_out(1, sem_o1)

    return k(table_t, idx)


def kernel(relative_position_bias_table, relative_position_index):
    rows, heads = relative_position_bias_table.shape
    stride = 4096
    table_t = jnp.pad(relative_position_bias_table.T,
                      ((0, 0), (0, stride - rows)))
    idx = relative_position_index.astype(jnp.int32)
    return _sc_gather(table_t, idx, heads, stride)


# speedup vs baseline: 1.0789x; 1.0789x over previous
import functools

import jax
import jax.numpy as jnp
from jax import lax
from jax.experimental import pallas as pl
from jax.experimental.pallas import tpu as pltpu
from jax.experimental.pallas import tpu_sc as plsc

NC, NS, L = 2, 16, 16
NW = NC * NS


@functools.partial(jax.jit, static_argnums=(2, 3))
def _sc_gather(table_t, idx, heads, stride):
    win2, win2b = idx.shape
    n = win2 * win2b
    per_tile_rows = win2 // NW
    b = win2b
    chunks = per_tile_rows
    groups = b // L
    hb = heads * b

    mesh = plsc.VectorSubcoreMesh(core_axis_name="c", subcore_axis_name="s")

    @functools.partial(
        pl.kernel,
        out_type=jax.ShapeDtypeStruct((heads, win2, win2b), jnp.float32),
        mesh=mesh,
        compiler_params=pltpu.CompilerParams(needs_layout_passes=False),
        scratch_types=[
            pltpu.VMEM((heads * stride,), jnp.float32),
            pltpu.VMEM((2 * b,), jnp.int32),
            pltpu.VMEM((2 * hb,), jnp.float32),
            pltpu.VMEM_SHARED((heads * stride,), jnp.float32),
            pltpu.SemaphoreType.DMA,
            pltpu.SemaphoreType.DMA,
            pltpu.SemaphoreType.DMA,
            pltpu.SemaphoreType.DMA,
            pltpu.SemaphoreType.DMA,
        ],
    )
    def k(tab_hbm, idx_hbm, out_hbm, tab_v, idx_v, out_v, tab_sh,
          sem_i0, sem_i1, sem_o0, sem_o1, sem_t):
        sid = lax.axis_index("s")
        wid = sid * NC + lax.axis_index("c")
        row0 = wid * per_tile_rows
        @pl.when(sid == 0)
        def _():
            for h in range(heads):
                pltpu.async_copy(tab_hbm.at[h],
                                 tab_sh.at[pl.ds(h * stride, stride)], sem_t)

        def idx_fetch(c, p, sem):
            pltpu.async_copy(idx_hbm.at[row0 + c],
                             idx_v.at[pl.ds(p * b, b)], sem)

        def drain_out(p, sem):
            for h in range(heads):
                pltpu.make_async_copy(out_hbm.at[0, 0],
                                      out_v.at[pl.ds(p * hb + h * b, b)],
                                      sem).wait()

        def do_chunk(c, p, sem_i, sem_o, first, last):
            pltpu.make_async_copy(idx_hbm.at[0],
                                  idx_v.at[pl.ds(p * b, b)], sem_i).wait()
            @pl.when(jnp.logical_not(first))
            def _():
                drain_out(p, sem_o)

            obase = p * hb
            ibase = p * b

            @plsc.parallel_loop(0, groups, unroll=4)
            def _(i):
                iv = idx_v[pl.ds(ibase + i * L, L)]
                for h in range(heads):
                    out_v[pl.ds(obase + h * b + i * L, L)] = (
                        plsc.load_gather(
                            tab_v.at[pl.ds(h * stride, stride)], [iv]))

            for h in range(heads):
                pltpu.async_copy(out_v.at[pl.ds(obase + h * b, b)],
                                 out_hbm.at[h, row0 + c], sem_o)

            @pl.when(jnp.logical_not(last))
            def _():
                idx_fetch(c + 2, p, sem_i)

        idx_fetch(0, 0, sem_i0)
        idx_fetch(1, 1, sem_i1)
        @pl.when(sid == 0)
        def _():
            for h in range(heads):
                pltpu.make_async_copy(tab_hbm.at[h],
                                      tab_sh.at[pl.ds(h * stride, stride)],
                                      sem_t).wait()
        plsc.subcore_barrier()
        pltpu.sync_copy(tab_sh, tab_v)

        n2 = chunks // 2

        @pl.loop(0, n2)
        def _(c2):
            c = 2 * c2
            do_chunk(c, 0, sem_i0, sem_o0, c2 == 0, c2 == n2 - 1)
            do_chunk(c + 1, 1, sem_i1, sem_o1, c2 == 0, c2 == n2 - 1)

        drain_out(0, sem_o0)
        drain_out(1, sem_o1)

    return k(table_t, idx)


def kernel(relative_position_bias_table, relative_position_index):
    rows, heads = relative_position_bias_table.shape
    stride = 4096
    table_t = jnp.pad(relative_position_bias_table.T,
                      ((0, 0), (0, stride - rows)))
    idx = relative_position_index.astype(jnp.int32)
    return _sc_gather(table_t, idx, heads, stride)
